# Initial kernel scaffold; baseline (speedup 1.0000x reference)
import functools

import jax
import jax.numpy as jnp
from jax import lax
from jax.experimental import pallas as pl
from jax.experimental.pallas import tpu as pltpu
from jax.experimental.pallas import tpu_sc as plsc

N_NODES = 10000
N_EDGES = 320000
DIM = 128
HEADS = 8
DH = DIM // HEADS
ATTN_SCALE = DH ** -0.5

NC = 2
NS = 16
NW = NC * NS
EDGES_PER_TILE = N_EDGES // NW
CHUNK = 400
ROWS_PER_TILE = N_NODES // NS

_BN = 2000
_BE = 2000


def _sc_mesh():
    return plsc.VectorSubcoreMesh(
        core_axis_name="c", subcore_axis_name="s", num_cores=NC, num_subcores=NS
    )


def _q_body(h_ref, wq_ref, bq_ref, o_ref):
    o_ref[...] = (
        jnp.dot(h_ref[...], wq_ref[...], preferred_element_type=jnp.float32)
        + bq_ref[...]
    ) * ATTN_SCALE


def _q_proj(h, Wq, bq2d):
    return pl.pallas_call(
        _q_body,
        grid=(N_NODES // _BN,),
        in_specs=[
            pl.BlockSpec((_BN, DIM), lambda i: (i, 0)),
            pl.BlockSpec((DIM, DIM), lambda i: (0, 0)),
            pl.BlockSpec((1, DIM), lambda i: (0, 0)),
        ],
        out_specs=pl.BlockSpec((_BN, DIM), lambda i: (i, 0)),
        out_shape=jax.ShapeDtypeStruct((N_NODES, DIM), jnp.float32),
    )(h, Wq, bq2d)


def _gather_qdst(q, dst):
    @functools.partial(
        pl.kernel,
        out_type=jax.ShapeDtypeStruct((N_EDGES, DIM), jnp.float32),
        mesh=_sc_mesh(),
        scratch_types=[
            pltpu.VMEM((CHUNK,), jnp.int32),
            pltpu.VMEM((CHUNK, DIM), jnp.float32),
            pltpu.SemaphoreType.DMA,
        ],
    )
    def gather_kernel(q_hbm, dst_hbm, out_hbm, idx_v, rows_v, sem):
        wid = lax.axis_index("c") * NS + lax.axis_index("s")
        base = wid * EDGES_PER_TILE

        @pl.loop(0, EDGES_PER_TILE, step=CHUNK)
        def _(off):
            pltpu.sync_copy(dst_hbm.at[pl.ds(base + off, CHUNK)], idx_v)
            pltpu.async_copy(q_hbm.at[idx_v], rows_v, sem).wait()
            pltpu.sync_copy(rows_v, out_hbm.at[pl.ds(base + off, CHUNK)])

    return gather_kernel(q, dst)


def _edge_body(e_ref, qd_ref, wkv_ref, bkv_ref, wmsg_ref, ex_ref):
    eb = e_ref[...]
    kv = (
        jnp.dot(eb, wkv_ref[...], preferred_element_type=jnp.float32)
        + bkv_ref[...]
    )
    k = kv[:, :DIM]
    v = kv[:, DIM:]
    t = qd_ref[...] * k
    lane = lax.broadcasted_iota(jnp.int32, (DIM, HEADS), 0)
    head = lax.broadcasted_iota(jnp.int32, (DIM, HEADS), 1)
    hsum_mat = (lane // DH == head).astype(jnp.float32)
    logit = jnp.dot(t, hsum_mat, preferred_element_type=jnp.float32)
    ex = jnp.exp(logit)
    head2 = lax.broadcasted_iota(jnp.int32, (HEADS, DIM), 0)
    lane2 = lax.broadcasted_iota(jnp.int32, (HEADS, DIM), 1)
    expand_mat = (lane2 // DH == head2).astype(jnp.float32)
    exb = jnp.dot(ex, expand_mat, preferred_element_type=jnp.float32)
    wmsg_ref[...] = exb * v
    ex_ref[...] = jnp.concatenate(
        [ex, jnp.zeros((ex.shape[0], DH - HEADS), jnp.float32)], axis=1
    )


def _edge_compute(e, qdst, Wkv, bkv2d):
    return pl.pallas_call(
        _edge_body,
        grid=(N_EDGES // _BE,),
        in_specs=[
            pl.BlockSpec((_BE, DIM), lambda i: (i, 0)),
            pl.BlockSpec((_BE, DIM), lambda i: (i, 0)),
            pl.BlockSpec((DIM, 2 * DIM), lambda i: (0, 0)),
            pl.BlockSpec((1, 2 * DIM), lambda i: (0, 0)),
        ],
        out_specs=[
            pl.BlockSpec((_BE, DIM), lambda i: (i, 0)),
            pl.BlockSpec((_BE, DH), lambda i: (i, 0)),
        ],
        out_shape=[
            jax.ShapeDtypeStruct((N_EDGES, DIM), jnp.float32),
            jax.ShapeDtypeStruct((N_EDGES, DH), jnp.float32),
        ],
    )(e, qdst, Wkv, bkv2d)


def _scatter(wmsg, ex16, dst):
    @functools.partial(
        pl.kernel,
        out_type=(
            jax.ShapeDtypeStruct((NC, N_NODES, DIM), jnp.float32),
            jax.ShapeDtypeStruct((NC, N_NODES, DH), jnp.float32),
        ),
        mesh=_sc_mesh(),
        scratch_types=[
            pltpu.VMEM((CHUNK,), jnp.int32),
            pltpu.VMEM((CHUNK, DIM), jnp.float32),
            pltpu.VMEM((CHUNK, DH), jnp.float32),
            pltpu.VMEM_SHARED((N_NODES, DIM), jnp.float32),
            pltpu.VMEM_SHARED((N_NODES, DH), jnp.float32),
        ],
    )
    def scatter_kernel(
        wmsg_hbm, ex_hbm, dst_hbm, outh_hbm, outd_hbm, idx_v, wbuf, exbuf, sh_h, sh_d
    ):
        cid = lax.axis_index("c")
        sid = lax.axis_index("s")

        @pl.loop(0, CHUNK)
        def _(i):
            for j in range(DIM // 16):
                wbuf[i, pl.ds(j * 16, 16)] = jnp.zeros((16,), jnp.float32)
            exbuf[i, pl.ds(0, 16)] = jnp.zeros((16,), jnp.float32)

        r0 = sid * ROWS_PER_TILE
        pltpu.sync_copy(wbuf.at[pl.ds(0, CHUNK)], sh_h.at[pl.ds(r0, CHUNK)])
        pltpu.sync_copy(
            wbuf.at[pl.ds(0, ROWS_PER_TILE - CHUNK)],
            sh_h.at[pl.ds(r0 + CHUNK, ROWS_PER_TILE - CHUNK)],
        )
        pltpu.sync_copy(exbuf.at[pl.ds(0, CHUNK)], sh_d.at[pl.ds(r0, CHUNK)])
        pltpu.sync_copy(
            exbuf.at[pl.ds(0, ROWS_PER_TILE - CHUNK)],
            sh_d.at[pl.ds(r0 + CHUNK, ROWS_PER_TILE - CHUNK)],
        )
        plsc.subcore_barrier()

        base = (cid * NS + sid) * EDGES_PER_TILE

        @pl.loop(0, EDGES_PER_TILE, step=CHUNK)
        def _(off):
            pltpu.sync_copy(dst_hbm.at[pl.ds(base + off, CHUNK)], idx_v)
            pltpu.sync_copy(wmsg_hbm.at[pl.ds(base + off, CHUNK)], wbuf)
            pltpu.sync_copy(ex_hbm.at[pl.ds(base + off, CHUNK)], exbuf)
            pltpu.sync_copy(wbuf, sh_h.at[idx_v], add=True)
            pltpu.sync_copy(exbuf, sh_d.at[idx_v], add=True)

        plsc.subcore_barrier()
        pltpu.sync_copy(
            sh_h.at[pl.ds(r0, ROWS_PER_TILE)],
            outh_hbm.at[cid].at[pl.ds(r0, ROWS_PER_TILE)],
        )
        pltpu.sync_copy(
            sh_d.at[pl.ds(r0, ROWS_PER_TILE)],
            outd_hbm.at[cid].at[pl.ds(r0, ROWS_PER_TILE)],
        )

    return scatter_kernel(wmsg, ex16, dst)


def _final_body(hsum_ref, denom_ref, wp_ref, bp_ref, o_ref):
    hs = hsum_ref[0] + hsum_ref[1]
    dn = denom_ref[0] + denom_ref[1]
    head = lax.broadcasted_iota(jnp.int32, (DH, DIM), 0)
    lane = lax.broadcasted_iota(jnp.int32, (DH, DIM), 1)
    expand_mat = (lane // DH == head).astype(jnp.float32)
    dn128 = jnp.dot(dn, expand_mat, preferred_element_type=jnp.float32)
    h_agg = jnp.where(dn128 > 0.0, hs / dn128, 0.0)
    o_ref[...] = (
        jnp.dot(h_agg, wp_ref[...], preferred_element_type=jnp.float32)
        + bp_ref[...]
    )


def _finalize(hsum, denom, Wp, bp2d):
    return pl.pallas_call(
        _final_body,
        grid=(N_NODES // _BN,),
        in_specs=[
            pl.BlockSpec((NC, _BN, DIM), lambda i: (0, i, 0)),
            pl.BlockSpec((NC, _BN, DH), lambda i: (0, i, 0)),
            pl.BlockSpec((DIM, DIM), lambda i: (0, 0)),
            pl.BlockSpec((1, DIM), lambda i: (0, 0)),
        ],
        out_specs=pl.BlockSpec((_BN, DIM), lambda i: (i, 0)),
        out_shape=jax.ShapeDtypeStruct((N_NODES, DIM), jnp.float32),
    )(hsum, denom, Wp, bp2d)


def kernel(h, e, edge_index, Wq, bq, Wkv, bkv, Wp, bp):
    dst = edge_index[1]
    q = _q_proj(h, Wq, bq.reshape(1, DIM))
    qdst = _gather_qdst(q, dst)
    wmsg, ex16 = _edge_compute(e, qdst, Wkv, bkv.reshape(1, 2 * DIM))
    hsum, denom = _scatter(wmsg, ex16, dst)
    return _finalize(hsum, denom, Wp, bp.reshape(1, DIM))


# speedup vs baseline: 35.5849x; 35.5849x over previous
import functools

import jax
import jax.numpy as jnp
from jax import lax
from jax.experimental import pallas as pl
from jax.experimental.pallas import tpu as pltpu
from jax.experimental.pallas import tpu_sc as plsc

N_NODES = 10000
N_EDGES = 320000
DIM = 128
HEADS = 8
DH = DIM // HEADS
ATTN_SCALE = DH ** -0.5

NC = 2
NS = 16
NW = NC * NS

HALF = DIM // 2
WIDTH = HALF + 16

GCHUNK = 40
GEDGES = N_EDGES // NW
SCHUNK = 80
SEDGES = N_EDGES // NS

ROWS_MAIN = 624
ROWS_TAIL = N_NODES - NS * ROWS_MAIN

_BN = 2000
_BE = 2000


def _sc_mesh():
    return plsc.VectorSubcoreMesh(
        core_axis_name="c", subcore_axis_name="s", num_cores=NC, num_subcores=NS
    )


def _gather_hdst(h, edge_index):
    @functools.partial(
        pl.kernel,
        out_type=jax.ShapeDtypeStruct((N_EDGES, DIM), jnp.float32),
        mesh=_sc_mesh(),
        scratch_types=[
            pltpu.VMEM((GCHUNK,), jnp.int32),
            pltpu.VMEM((GCHUNK, DIM), jnp.float32),
        ],
    )
    def gather_kernel(h_hbm, ei_hbm, out_hbm, idx_v, rows_v):
        wid = lax.axis_index("c") * NS + lax.axis_index("s")
        base = N_EDGES + wid * GEDGES

        @pl.loop(0, GEDGES, step=GCHUNK)
        def _(off):
            pltpu.sync_copy(ei_hbm.at[pl.ds(base + off, GCHUNK)], idx_v)
            pltpu.sync_copy(h_hbm.at[idx_v], rows_v)
            pltpu.sync_copy(rows_v, out_hbm.at[pl.ds(base + off, GCHUNK)])

    return gather_kernel(h, edge_index)


def _edge_body(e_ref, hd_ref, wq_ref, bq_ref, wkv_ref, bkv_ref, o_ref):
    qd = (
        jnp.dot(hd_ref[...], wq_ref[...], preferred_element_type=jnp.float32)
        + bq_ref[...]
    ) * ATTN_SCALE
    kv = (
        jnp.dot(e_ref[...], wkv_ref[...], preferred_element_type=jnp.float32)
        + bkv_ref[...]
    )
    k = kv[:, :DIM]
    v = kv[:, DIM:]
    t = qd * k
    lane = lax.broadcasted_iota(jnp.int32, (DIM, HEADS), 0)
    head = lax.broadcasted_iota(jnp.int32, (DIM, HEADS), 1)
    hsum_mat = (lane // DH == head).astype(jnp.float32)
    logit = jnp.dot(t, hsum_mat, preferred_element_type=jnp.float32)
    ex = jnp.exp(logit)
    head2 = lax.broadcasted_iota(jnp.int32, (HEADS, DIM), 0)
    lane2 = lax.broadcasted_iota(jnp.int32, (HEADS, DIM), 1)
    expand_mat = (lane2 // DH == head2).astype(jnp.float32)
    exb = jnp.dot(ex, expand_mat, preferred_element_type=jnp.float32)
    wm = exb * v
    expad = jnp.concatenate(
        [ex, jnp.zeros((ex.shape[0], 16 - HEADS), jnp.float32)], axis=1
    )
    row0 = jnp.concatenate([wm[:, :HALF], expad], axis=1)
    row1 = jnp.concatenate([wm[:, HALF:], expad], axis=1)
    o_ref[...] = jnp.stack([row0, row1], axis=0)


def _edge_compute(e, hdst, Wq, bq2d, Wkv, bkv2d):
    return pl.pallas_call(
        _edge_body,
        grid=(N_EDGES // _BE,),
        in_specs=[
            pl.BlockSpec((_BE, DIM), lambda i: (i, 0)),
            pl.BlockSpec((_BE, DIM), lambda i: (i, 0)),
            pl.BlockSpec((DIM, DIM), lambda i: (0, 0)),
            pl.BlockSpec((1, DIM), lambda i: (0, 0)),
            pl.BlockSpec((DIM, 2 * DIM), lambda i: (0, 0)),
            pl.BlockSpec((1, 2 * DIM), lambda i: (0, 0)),
        ],
        out_specs=pl.BlockSpec((NC, _BE, WIDTH), lambda i: (0, i, 0)),
        out_shape=jax.ShapeDtypeStruct((NC, N_EDGES, WIDTH), jnp.float32),
    )(e, hdst, Wq, bq2d, Wkv, bkv2d)


def _scatter(msg, edge_index):
    @functools.partial(
        pl.kernel,
        out_type=jax.ShapeDtypeStruct((NC, N_NODES, WIDTH), jnp.float32),
        mesh=_sc_mesh(),
        scratch_types=[
            pltpu.VMEM((SCHUNK,), jnp.int32),
            pltpu.VMEM((SCHUNK, WIDTH), jnp.float32),
            pltpu.VMEM_SHARED((N_NODES, WIDTH), jnp.float32),
        ],
    )
    def scatter_kernel(msg_hbm, ei_hbm, out_hbm, idx_v, buf, sh):
        cid = lax.axis_index("c")
        sid = lax.axis_index("s")

        @pl.loop(0, SCHUNK)
        def _(i):
            for j in range(WIDTH // 16):
                buf[i, pl.ds(j * 16, 16)] = jnp.zeros((16,), jnp.float32)

        r0 = sid * ROWS_MAIN

        @pl.loop(0, ROWS_MAIN - SCHUNK, step=SCHUNK)
        def _(r):
            pltpu.sync_copy(buf, sh.at[pl.ds(r0 + r, SCHUNK)])

        rem = ROWS_MAIN % SCHUNK if ROWS_MAIN % SCHUNK else SCHUNK
        pltpu.sync_copy(
            buf.at[pl.ds(0, rem)], sh.at[pl.ds(r0 + ROWS_MAIN - rem, rem)]
        )

        @pl.when(sid == NS - 1)
        def _():
            t0 = NS * ROWS_MAIN
            pltpu.sync_copy(buf.at[pl.ds(0, ROWS_TAIL)], sh.at[pl.ds(t0, ROWS_TAIL)])

        plsc.subcore_barrier()

        ebase = N_EDGES + sid * SEDGES

        @pl.loop(0, SEDGES, step=SCHUNK)
        def _(off):
            pltpu.sync_copy(ei_hbm.at[pl.ds(ebase + off, SCHUNK)], idx_v)
            pltpu.sync_copy(msg_hbm.at[cid].at[pl.ds(ebase + off, SCHUNK)], buf)
            pltpu.sync_copy(buf, sh.at[idx_v], add=True)

        plsc.subcore_barrier()
        pltpu.sync_copy(
            sh.at[pl.ds(r0, ROWS_MAIN)], out_hbm.at[cid].at[pl.ds(r0, ROWS_MAIN)]
        )

        @pl.when(sid == NS - 1)
        def _():
            t0 = NS * ROWS_MAIN
            pltpu.sync_copy(
                sh.at[pl.ds(t0, ROWS_TAIL)], out_hbm.at[cid].at[pl.ds(t0, ROWS_TAIL)]
            )

    return scatter_kernel(msg, edge_index)


def _final_body(acc_ref, wp_ref, bp_ref, o_ref):
    hs = jnp.concatenate(
        [acc_ref[0][:, :HALF], acc_ref[1][:, :HALF]], axis=1
    )
    dn = acc_ref[0][:, HALF : HALF + HEADS]
    head = lax.broadcasted_iota(jnp.int32, (HEADS, DIM), 0)
    lane = lax.broadcasted_iota(jnp.int32, (HEADS, DIM), 1)
    expand_mat = (lane // DH == head).astype(jnp.float32)
    dn128 = jnp.dot(dn, expand_mat, preferred_element_type=jnp.float32)
    h_agg = jnp.where(dn128 > 0.0, hs / dn128, 0.0)
    o_ref[...] = (
        jnp.dot(h_agg, wp_ref[...], preferred_element_type=jnp.float32)
        + bp_ref[...]
    )


def _finalize(acc, Wp, bp2d):
    return pl.pallas_call(
        _final_body,
        grid=(N_NODES // _BN,),
        in_specs=[
            pl.BlockSpec((NC, _BN, WIDTH), lambda i: (0, i, 0)),
            pl.BlockSpec((DIM, DIM), lambda i: (0, 0)),
            pl.BlockSpec((1, DIM), lambda i: (0, 0)),
        ],
        out_specs=pl.BlockSpec((_BN, DIM), lambda i: (i, 0)),
        out_shape=jax.ShapeDtypeStruct((N_NODES, DIM), jnp.float32),
    )(acc, Wp, bp2d)


def kernel(h, e, edge_index, Wq, bq, Wkv, bkv, Wp, bp):
    ei_flat = edge_index.reshape(-1)
    hdst = _gather_hdst(h, ei_flat)
    msg = _edge_compute(
        e, hdst, Wq, bq.reshape(1, DIM), Wkv, bkv.reshape(1, 2 * DIM)
    )
    acc = _scatter(msg, ei_flat)
    return _finalize(acc, Wp, bp.reshape(1, DIM))


# speedup vs baseline: 41.8757x; 1.1768x over previous
import functools

import jax
import jax.numpy as jnp
from jax import lax
from jax.experimental import pallas as pl
from jax.experimental.pallas import tpu as pltpu
from jax.experimental.pallas import tpu_sc as plsc

N_NODES = 10000
N_EDGES = 320000
DIM = 128
HEADS = 8
DH = DIM // HEADS
ATTN_SCALE = DH ** -0.5

NC = 2
NS = 16
NW = NC * NS

HALF = DIM // 2
WIDTH = HALF + 16

GCHUNK = 40
GEDGES = N_EDGES // NW
SCHUNK = 40
SEDGES = N_EDGES // NS

ROWS_MAIN = 624
ROWS_TAIL = N_NODES - NS * ROWS_MAIN

_BN = 2000
_BE = 2000


def _sc_mesh():
    return plsc.VectorSubcoreMesh(
        core_axis_name="c", subcore_axis_name="s", num_cores=NC, num_subcores=NS
    )


def _gather_hdst(h, edge_index):
    @functools.partial(
        pl.kernel,
        out_type=jax.ShapeDtypeStruct((N_EDGES, DIM), jnp.float32),
        mesh=_sc_mesh(),
        scratch_types=[
            pltpu.VMEM((GCHUNK,), jnp.int32),
            pltpu.VMEM((GCHUNK, DIM), jnp.float32),
        ],
    )
    def gather_kernel(h_hbm, ei_hbm, out_hbm, idx_v, rows_v):
        wid = lax.axis_index("c") * NS + lax.axis_index("s")
        base = N_EDGES + wid * GEDGES

        @pl.loop(0, GEDGES, step=GCHUNK)
        def _(off):
            pltpu.sync_copy(ei_hbm.at[pl.ds(base + off, GCHUNK)], idx_v)
            pltpu.sync_copy(h_hbm.at[idx_v], rows_v)
            pltpu.sync_copy(rows_v, out_hbm.at[pl.ds(base + off, GCHUNK)])

    return gather_kernel(h, edge_index)


def _edge_body(e_ref, hd_ref, wq_ref, bq_ref, wkv_ref, bkv_ref, o_ref):
    qd = (
        jnp.dot(hd_ref[...], wq_ref[...], preferred_element_type=jnp.float32)
        + bq_ref[...]
    ) * ATTN_SCALE
    kv = (
        jnp.dot(e_ref[...], wkv_ref[...], preferred_element_type=jnp.float32)
        + bkv_ref[...]
    )
    k = kv[:, :DIM]
    v = kv[:, DIM:]
    t = qd * k
    lane = lax.broadcasted_iota(jnp.int32, (DIM, HEADS), 0)
    head = lax.broadcasted_iota(jnp.int32, (DIM, HEADS), 1)
    hsum_mat = (lane // DH == head).astype(jnp.float32)
    logit = jnp.dot(t, hsum_mat, preferred_element_type=jnp.float32)
    ex = jnp.exp(logit)
    head2 = lax.broadcasted_iota(jnp.int32, (HEADS, DIM), 0)
    lane2 = lax.broadcasted_iota(jnp.int32, (HEADS, DIM), 1)
    expand_mat = (lane2 // DH == head2).astype(jnp.float32)
    exb = jnp.dot(ex, expand_mat, preferred_element_type=jnp.float32)
    wm = exb * v
    expad = jnp.concatenate(
        [ex, jnp.zeros((ex.shape[0], 16 - HEADS), jnp.float32)], axis=1
    )
    row0 = jnp.concatenate([wm[:, :HALF], expad], axis=1)
    row1 = jnp.concatenate([wm[:, HALF:], expad], axis=1)
    o_ref[...] = jnp.stack([row0, row1], axis=0)


def _edge_compute(e, hdst, Wq, bq2d, Wkv, bkv2d):
    return pl.pallas_call(
        _edge_body,
        grid=(N_EDGES // _BE,),
        in_specs=[
            pl.BlockSpec((_BE, DIM), lambda i: (i, 0)),
            pl.BlockSpec((_BE, DIM), lambda i: (i, 0)),
            pl.BlockSpec((DIM, DIM), lambda i: (0, 0)),
            pl.BlockSpec((1, DIM), lambda i: (0, 0)),
            pl.BlockSpec((DIM, 2 * DIM), lambda i: (0, 0)),
            pl.BlockSpec((1, 2 * DIM), lambda i: (0, 0)),
        ],
        out_specs=pl.BlockSpec((NC, _BE, WIDTH), lambda i: (0, i, 0)),
        out_shape=jax.ShapeDtypeStruct((NC, N_EDGES, WIDTH), jnp.float32),
    )(e, hdst, Wq, bq2d, Wkv, bkv2d)


def _scatter(msg, edge_index):
    @functools.partial(
        pl.kernel,
        out_type=jax.ShapeDtypeStruct((NC, N_NODES, WIDTH), jnp.float32),
        mesh=_sc_mesh(),
        scratch_types=[
            pltpu.VMEM((2, SCHUNK), jnp.int32),
            pltpu.VMEM((2, SCHUNK, WIDTH), jnp.float32),
            pltpu.VMEM_SHARED((N_NODES, WIDTH), jnp.float32),
            pltpu.SemaphoreType.DMA,
            pltpu.SemaphoreType.DMA,
        ],
    )
    def scatter_kernel(msg_hbm, ei_hbm, out_hbm, idx2, buf2, sh, sem0, sem1):
        cid = lax.axis_index("c")
        sid = lax.axis_index("s")
        sems = (sem0, sem1)
        buf = buf2.at[0]
        idx_v = idx2.at[0]

        @pl.loop(0, SCHUNK)
        def _(i):
            for j in range(WIDTH // 16):
                buf[i, pl.ds(j * 16, 16)] = jnp.zeros((16,), jnp.float32)

        r0 = sid * ROWS_MAIN

        @pl.loop(0, ROWS_MAIN - SCHUNK, step=SCHUNK)
        def _(r):
            pltpu.sync_copy(buf, sh.at[pl.ds(r0 + r, SCHUNK)])

        rem = ROWS_MAIN % SCHUNK if ROWS_MAIN % SCHUNK else SCHUNK
        pltpu.sync_copy(
            buf.at[pl.ds(0, rem)], sh.at[pl.ds(r0 + ROWS_MAIN - rem, rem)]
        )

        @pl.when(sid == NS - 1)
        def _():
            t0 = NS * ROWS_MAIN
            pltpu.sync_copy(buf.at[pl.ds(0, ROWS_TAIL)], sh.at[pl.ds(t0, ROWS_TAIL)])

        plsc.subcore_barrier()

        ebase = N_EDGES + sid * SEDGES
        mbase = sid * SEDGES

        for b in range(2):
            pltpu.async_copy(
                ei_hbm.at[pl.ds(ebase + b * SCHUNK, SCHUNK)], idx2.at[b], sems[b]
            )
            pltpu.async_copy(
                msg_hbm.at[cid].at[pl.ds(mbase + b * SCHUNK, SCHUNK)],
                buf2.at[b],
                sems[b],
            )

        plsc.subcore_barrier()

        @pl.loop(0, SEDGES - 2 * SCHUNK, step=2 * SCHUNK)
        def _(off):
            for b in range(2):
                pltpu.make_async_copy(
                    ei_hbm.at[pl.ds(ebase, SCHUNK)], idx2.at[b], sems[b]
                ).wait()
                pltpu.make_async_copy(
                    msg_hbm.at[cid].at[pl.ds(mbase, SCHUNK)], buf2.at[b], sems[b]
                ).wait()
                pltpu.sync_copy(buf2.at[b], sh.at[idx2.at[b]], add=True)
                nxt = off + (b + 2) * SCHUNK
                pltpu.async_copy(
                    ei_hbm.at[pl.ds(ebase + nxt, SCHUNK)], idx2.at[b], sems[b]
                )
                pltpu.async_copy(
                    msg_hbm.at[cid].at[pl.ds(mbase + nxt, SCHUNK)],
                    buf2.at[b],
                    sems[b],
                )

        for b in range(2):
            pltpu.make_async_copy(
                ei_hbm.at[pl.ds(ebase, SCHUNK)], idx2.at[b], sems[b]
            ).wait()
            pltpu.make_async_copy(
                msg_hbm.at[cid].at[pl.ds(mbase, SCHUNK)], buf2.at[b], sems[b]
            ).wait()
            pltpu.sync_copy(buf2.at[b], sh.at[idx2.at[b]], add=True)

        plsc.subcore_barrier()
        pltpu.sync_copy(
            sh.at[pl.ds(r0, ROWS_MAIN)], out_hbm.at[cid].at[pl.ds(r0, ROWS_MAIN)]
        )

        @pl.when(sid == NS - 1)
        def _():
            t0 = NS * ROWS_MAIN
            pltpu.sync_copy(
                sh.at[pl.ds(t0, ROWS_TAIL)], out_hbm.at[cid].at[pl.ds(t0, ROWS_TAIL)]
            )

    return scatter_kernel(msg, edge_index)


def _final_body(acc_ref, wp_ref, bp_ref, o_ref):
    hs = jnp.concatenate(
        [acc_ref[0][:, :HALF], acc_ref[1][:, :HALF]], axis=1
    )
    dn = acc_ref[0][:, HALF : HALF + HEADS]
    head = lax.broadcasted_iota(jnp.int32, (HEADS, DIM), 0)
    lane = lax.broadcasted_iota(jnp.int32, (HEADS, DIM), 1)
    expand_mat = (lane // DH == head).astype(jnp.float32)
    dn128 = jnp.dot(dn, expand_mat, preferred_element_type=jnp.float32)
    h_agg = jnp.where(dn128 > 0.0, hs / dn128, 0.0)
    o_ref[...] = (
        jnp.dot(h_agg, wp_ref[...], preferred_element_type=jnp.float32)
        + bp_ref[...]
    )


def _finalize(acc, Wp, bp2d):
    return pl.pallas_call(
        _final_body,
        grid=(N_NODES // _BN,),
        in_specs=[
            pl.BlockSpec((NC, _BN, WIDTH), lambda i: (0, i, 0)),
            pl.BlockSpec((DIM, DIM), lambda i: (0, 0)),
            pl.BlockSpec((1, DIM), lambda i: (0, 0)),
        ],
        out_specs=pl.BlockSpec((_BN, DIM), lambda i: (i, 0)),
        out_shape=jax.ShapeDtypeStruct((N_NODES, DIM), jnp.float32),
    )(acc, Wp, bp2d)


def kernel(h, e, edge_index, Wq, bq, Wkv, bkv, Wp, bp):
    ei_flat = edge_index.reshape(-1)
    hdst = _gather_hdst(h, ei_flat)
    msg = _edge_compute(
        e, hdst, Wq, bq.reshape(1, DIM), Wkv, bkv.reshape(1, 2 * DIM)
    )
    acc = _scatter(msg, ei_flat)
    return _finalize(acc, Wp, bp.reshape(1, DIM))
